# baseline (speedup 1.0000x reference)
import jax
import jax.numpy as jnp
from jax import lax
from jax.experimental import pallas as pl
from jax.experimental.pallas import tpu as pltpu
from jax.experimental.pallas import tpu_sc as plsc

VAR_LEN = 1000000
EMBED_SIZE = 64
BATCH = 16384
FIELDS = 26

NUM_CORES = 2
NUM_SUBCORES = 16
NUM_WORKERS = NUM_CORES * NUM_SUBCORES

B_TOTAL = BATCH * FIELDS
B_PER_W = B_TOTAL // NUM_WORKERS
CHUNK = 512
N_CHUNKS = B_PER_W // CHUNK


def _emb_body(idx_hbm, table_hbm, out_hbm, idx_v, rows_v, sem):
    wid = lax.axis_index("s") * NUM_CORES + lax.axis_index("c")
    base = pl.multiple_of(wid * B_PER_W, B_PER_W)
    pltpu.sync_copy(idx_hbm.at[pl.ds(base, B_PER_W)], idx_v)

    def step(j, carry):
        off = pl.multiple_of(j * CHUNK, CHUNK)
        pltpu.async_copy(table_hbm.at[idx_v.at[pl.ds(off, CHUNK)]], rows_v, sem).wait()
        pltpu.sync_copy(rows_v, out_hbm.at[pl.ds(base + off, CHUNK)])
        return carry

    lax.fori_loop(0, N_CHUNKS, step, 0)


@jax.jit
def _emb(idx_flat, weight):
    mesh = plsc.VectorSubcoreMesh(core_axis_name="c", subcore_axis_name="s")
    return pl.kernel(
        _emb_body,
        out_type=jax.ShapeDtypeStruct((B_TOTAL, EMBED_SIZE), jnp.float32),
        mesh=mesh,
        scratch_types=[
            pltpu.VMEM((B_PER_W,), jnp.int32),
            pltpu.VMEM((CHUNK, EMBED_SIZE), jnp.float32),
            pltpu.SemaphoreType.DMA,
        ],
        compiler_params=pltpu.CompilerParams(use_tc_tiling_on_sc=False),
    )(idx_flat, weight)


def kernel(input, weight):
    idx_flat = input.reshape(-1).astype(jnp.int32)
    out = _emb(idx_flat, weight)
    return out.reshape(BATCH, FIELDS, EMBED_SIZE)


# speedup vs baseline: 1.0174x; 1.0174x over previous
import jax
import jax.numpy as jnp
from jax import lax
from jax.experimental import pallas as pl
from jax.experimental.pallas import tpu as pltpu
from jax.experimental.pallas import tpu_sc as plsc

VAR_LEN = 1000000
EMBED_SIZE = 64
BATCH = 16384
FIELDS = 26

NUM_CORES = 2
NUM_SUBCORES = 16
NUM_WORKERS = NUM_CORES * NUM_SUBCORES

B_TOTAL = BATCH * FIELDS
B_PER_W = B_TOTAL // NUM_WORKERS
CHUNK = 208
N_CHUNKS = B_PER_W // CHUNK
NBUF = 8
LOOKAHEAD = 4
N_BLOCKS = N_CHUNKS // NBUF


def _emb_body(idx_hbm, table_hbm, out_hbm, idx_v, rows_v, gsems, wsems):
    wid = lax.axis_index("s") * NUM_CORES + lax.axis_index("c")
    base = pl.multiple_of(wid * B_PER_W, B_PER_W)
    pltpu.sync_copy(idx_hbm.at[pl.ds(base, B_PER_W)], idx_v)

    def gather_start(j, b):
        off = pl.multiple_of(j * CHUNK, CHUNK)
        pltpu.make_async_copy(
            table_hbm.at[idx_v.at[pl.ds(off, CHUNK)]], rows_v.at[b], gsems.at[b]
        ).start()

    def gather_wait(b):
        pltpu.make_async_copy(
            table_hbm.at[idx_v.at[pl.ds(0, CHUNK)]], rows_v.at[b], gsems.at[b]
        ).wait()

    def write_start(j, b):
        off = pl.multiple_of(j * CHUNK, CHUNK)
        pltpu.make_async_copy(
            rows_v.at[b], out_hbm.at[pl.ds(base + off, CHUNK)], wsems.at[b]
        ).start()

    def write_wait(b):
        pltpu.make_async_copy(
            rows_v.at[b], out_hbm.at[pl.ds(base, CHUNK)], wsems.at[b]
        ).wait()

    for b in range(LOOKAHEAD):
        gather_start(b, b)

    def run_chunk(j, b, fire, drain):
        fb = (b + LOOKAHEAD) % NBUF
        if fire:
            if drain:
                write_wait(fb)
            gather_start(j + LOOKAHEAD, fb)
        gather_wait(b)
        write_start(j, b)

    for b in range(NBUF):
        run_chunk(b, b, fire=True, drain=(b + LOOKAHEAD >= NBUF))

    def block(gi, carry):
        g = gi * NBUF
        for b in range(NBUF):
            run_chunk(g + b, b, fire=True, drain=True)
        return carry

    lax.fori_loop(1, N_BLOCKS - 1, block, 0)

    g = N_CHUNKS - NBUF
    for b in range(NBUF):
        run_chunk(g + b, b, fire=(b + LOOKAHEAD < NBUF), drain=(b + LOOKAHEAD < NBUF))

    for b in range(NBUF):
        write_wait(b)


@jax.jit
def _emb(idx_flat, weight):
    mesh = plsc.VectorSubcoreMesh(core_axis_name="c", subcore_axis_name="s")
    return pl.kernel(
        _emb_body,
        out_type=jax.ShapeDtypeStruct((B_TOTAL, EMBED_SIZE), jnp.float32),
        mesh=mesh,
        scratch_types=[
            pltpu.VMEM((B_PER_W,), jnp.int32),
            pltpu.VMEM((NBUF, CHUNK, EMBED_SIZE), jnp.float32),
            pltpu.SemaphoreType.DMA((NBUF,)),
            pltpu.SemaphoreType.DMA((NBUF,)),
        ],
        compiler_params=pltpu.CompilerParams(use_tc_tiling_on_sc=False),
    )(idx_flat, weight)


def kernel(input, weight):
    idx_flat = input.reshape(-1).astype(jnp.int32)
    out = _emb(idx_flat, weight)
    return out.reshape(BATCH, FIELDS, EMBED_SIZE)
